# baseline (speedup 1.0000x reference)
import functools

import numpy as np
import jax
import jax.numpy as jnp
from jax import lax
from jax.experimental import pallas as pl
from jax.experimental.pallas import tpu as pltpu

_EPS = 1e-5
_F32 = jnp.float32


def _bn_ab(bn):
    g, b, m, v = bn
    inv = g / jnp.sqrt(v + _EPS)
    return inv[None, :], (b - m * inv)[None, :]


def _w3(w):
    o, i = w.shape[0], w.shape[1]
    return jnp.transpose(w, (2, 3, 1, 0)).reshape(9 * i, o)


def _w1(w):
    return jnp.transpose(w[:, :, 0, 0])


def _ws2(w):
    o, i = w.shape[0], w.shape[1]
    wt = jnp.transpose(w, (2, 3, 1, 0))
    we = jnp.zeros((2, 2, 2, 2, i, o), _F32)
    for dy in range(3):
        for dx in range(3):
            we = we.at[dy // 2, dx // 2, dy % 2, dx % 2].set(wt[dy, dx])
    return we.reshape(16 * i, o)


def _fc_fold(fc_w):
    iu = np.triu_indices(64)
    wf = jnp.zeros((64, 64, 128), _F32)
    wf = wf.at[iu[0], iu[1], :].set(jnp.transpose(fc_w))
    return jnp.transpose(wf, (1, 0, 2)).reshape(64, 64 * 128)


def _s2d(y):
    b, h, w, c = y.shape
    return (y.reshape(b, h // 2, 2, w // 2, 2, c)
             .transpose(0, 1, 3, 2, 4, 5)
             .reshape(b, h // 2, w // 2, 4 * c))


def _relu(x):
    return jnp.maximum(x, 0.0)


def _mmc(a, b):
    return jnp.dot(a.astype(jnp.bfloat16), b.astype(jnp.bfloat16),
                   preferred_element_type=_F32)


def _mmh(a, b):
    return jnp.dot(a, b, preferred_element_type=_F32,
                   precision=lax.Precision.HIGHEST)


def _conv3(x, wref):
    h, w, c = x.shape
    xp = jnp.pad(x, ((1, 1), (1, 1), (0, 0)))
    cols = jnp.concatenate(
        [xp[dy:dy + h, dx:dx + w, :] for dy in range(3) for dx in range(3)],
        axis=2)
    return _mmc(cols.reshape(h * w, 9 * c), wref[:])


def _conv_s2(x, wref):
    h, w, c = x.shape
    xp = jnp.pad(x, ((0, 1), (0, 1), (0, 0)))
    cols = jnp.concatenate(
        [xp[t:t + h, s:s + w, :] for t in range(2) for s in range(2)],
        axis=2)
    return _mmc(cols.reshape(h * w, 4 * c), wref[:])


def _sqrtm_ns(a, n):
    i0 = lax.broadcasted_iota(jnp.int32, (n, n), 0)
    i1 = lax.broadcasted_iota(jnp.int32, (n, n), 1)
    eye = jnp.where(i0 == i1, 1.0, 0.0).astype(_F32)
    i3 = 3.0 * eye
    tr = jnp.sum(a * eye, axis=(0, 1), keepdims=True)
    an = a / tr
    zy = 0.5 * (i3 - an)
    y = _mmh(an, zy)
    z = zy
    for _ in range(3):
        t = 0.5 * (i3 - _mmh(z, y))
        y, z = _mmh(y, t), _mmh(t, z)
    yzy = 0.5 * _mmh(y, i3 - _mmh(z, y))
    return yzy * jnp.sqrt(tr)


def _cbr(x, wref, ab, relu=True):
    y = _conv3(x, wref) * ab[0][:] + ab[1][:]
    return _relu(y) if relu else y


def _stage1_body(x_ref, wc1, a0, b0, w11, a1, b1, w12, a2, b2,
                 w21, a3, b3, w22, a4, b4, out_ref):
    x = x_ref[0]
    h = _cbr(x, wc1, (a0, b0))
    t = _cbr(h.reshape(64, 64, 16), w11, (a1, b1))
    t = _cbr(t.reshape(64, 64, 16), w12, (a2, b2), relu=False)
    h = _relu(t + h)
    t = _cbr(h.reshape(64, 64, 16), w21, (a3, b3))
    t = _cbr(t.reshape(64, 64, 16), w22, (a4, b4), relu=False)
    h = _relu(t + h)
    out_ref[0] = h.reshape(64, 64, 16)


def _mid_body(x_ref, ws2, a0, b0, wd, ad, bd, w12, a2, b2,
              w21, a3, b3, w22, a4, b4, out_ref, *, s, c):
    x = x_ref[0]
    m = s * s
    cin4 = x.shape[2]
    c1 = _relu(_conv_s2(x, ws2) * a0[:] + b0[:])
    res = _mmc(x.reshape(m, cin4)[:, :cin4 // 4], wd[:]) * ad[:] + bd[:]
    t = _cbr(c1.reshape(s, s, c), w12, (a2, b2), relu=False)
    h = _relu(t + res)
    t = _cbr(h.reshape(s, s, c), w21, (a3, b3))
    t = _cbr(t.reshape(s, s, c), w22, (a4, b4), relu=False)
    h = _relu(t + h)
    out_ref[0] = h.reshape(s, s, c)


def _stage4_body(x_ref, ws2, a0, b0, wd, ad, bd, w12, a2, b2,
                 w21, a3, b3, w22, a4, b4,
                 wred, ared, bred, watt, aatt, batt, wfc, bfc, out_ref):
    x = x_ref[0]
    c1 = _relu(_conv_s2(x, ws2) * a0[:] + b0[:])
    res = _mmc(x.reshape(64, 256)[:, :64], wd[:]) * ad[:] + bd[:]
    t = _cbr(c1.reshape(8, 8, 128), w12, (a2, b2), relu=False)
    h = _relu(t + res)
    t = _cbr(h.reshape(8, 8, 128), w21, (a3, b3))
    t = _cbr(t.reshape(8, 8, 128), w22, (a4, b4), relu=False)
    h = _relu(t + h)
    z = _relu(_mmc(h, wred[:]) * ared[:] + bred[:])
    a = _relu(_mmc(z, watt[:]) * aatt[:] + batt[:])
    ac = a - jnp.mean(a, axis=0, keepdims=True)
    cov = lax.dot_general(ac, ac, (((0,), (0,)), ((), ())),
                          preferred_element_type=_F32,
                          precision=lax.Precision.HIGHEST) * (1.0 / 64.0)
    ya = _sqrtm_ns(cov, 64)
    u = _mmh(ya, wfc[:])
    parts = [u[i:i + 1, 128 * i:128 * (i + 1)] for i in range(64)]
    while len(parts) > 1:
        parts = [parts[i] + parts[i + 1] for i in range(0, len(parts), 2)]
    gate = jax.nn.sigmoid(parts[0] + bfc[:])
    zg = z * gate
    zc = zg - jnp.mean(zg, axis=0, keepdims=True)
    cov2 = lax.dot_general(zc, zc, (((0,), (0,)), ((), ())),
                           preferred_element_type=_F32,
                           precision=lax.Precision.HIGHEST) * (1.0 / 64.0)
    out_ref[0] = _sqrtm_ns(cov2, 128)


def _full_spec(shape):
    nd = len(shape)
    return pl.BlockSpec(shape, lambda b, _nd=nd: (0,) * _nd)


def _item_spec(shape):
    nd = len(shape)
    return pl.BlockSpec((1,) + shape,
                        lambda b, _nd=nd: (b,) + (0,) * _nd)


def _run_stage(body, x, ws, out_elem_shape):
    b = x.shape[0]
    in_specs = [_item_spec(x.shape[1:])] + [_full_spec(w.shape) for w in ws]
    return pl.pallas_call(
        body,
        grid=(b,),
        in_specs=in_specs,
        out_specs=_item_spec(out_elem_shape),
        out_shape=jax.ShapeDtypeStruct((b,) + out_elem_shape, _F32),
        compiler_params=pltpu.CompilerParams(
            dimension_semantics=("parallel",),
            vmem_limit_bytes=56 * 1024 * 1024,
        ),
    )(x, *ws)


def kernel(x, params):
    p = params
    b = x.shape[0]
    l1, l2, l3, l4 = p['layers']
    xh = jnp.transpose(x, (0, 2, 3, 1))

    ws1 = [_w3(p['conv1']), *_bn_ab(p['bn1'])]
    for blk in l1:
        for wk, bk in (('w1', 'bn1'), ('w2', 'bn2')):
            ws1 += [_w3(blk[wk]), *_bn_ab(blk[bk])]
    a1 = _run_stage(_stage1_body, xh, ws1, (64, 64, 16))

    def mid_ws(blk0, blk1):
        ws = [_ws2(blk0['w1']), *_bn_ab(blk0['bn1'])]
        ws += [_w1(blk0['dw']), *_bn_ab(blk0['dbn'])]
        ws += [_w3(blk0['w2']), *_bn_ab(blk0['bn2'])]
        for wk, bk in (('w1', 'bn1'), ('w2', 'bn2')):
            ws += [_w3(blk1[wk]), *_bn_ab(blk1[bk])]
        return ws

    a2 = _run_stage(functools.partial(_mid_body, s=32, c=32),
                    _s2d(a1), mid_ws(l2[0], l2[1]), (32, 32, 32))
    a3 = _run_stage(functools.partial(_mid_body, s=16, c=64),
                    _s2d(a2), mid_ws(l3[0], l3[1]), (16, 16, 64))

    ws4 = mid_ws(l4[0], l4[1])
    ws4 += [_w1(p['reduce_w']), *_bn_ab(p['reduce_bn'])]
    ws4 += [_w1(p['att']['w']), *_bn_ab(p['att']['bn'])]
    ws4 += [_fc_fold(p['att']['fc_w']), p['att']['fc_b'][None, :]]
    y = _run_stage(_stage4_body, _s2d(a3), ws4, (128, 128))

    iu = np.triu_indices(128)
    flat = jnp.asarray(iu[0] * 128 + iu[1], jnp.int32)
    return y.reshape(b, 128 * 128)[:, flat]


# speedup vs baseline: 1.0413x; 1.0413x over previous
import functools

import numpy as np
import jax
import jax.numpy as jnp
from jax import lax
from jax.experimental import pallas as pl
from jax.experimental.pallas import tpu as pltpu

_EPS = 1e-5
_F32 = jnp.float32


def _bn_ab(bn):
    g, b, m, v = bn
    inv = g / jnp.sqrt(v + _EPS)
    return inv[None, :], (b - m * inv)[None, :]


def _w3(w):
    o, i = w.shape[0], w.shape[1]
    return jnp.transpose(w, (2, 3, 1, 0)).reshape(9 * i, o)


def _w1(w):
    return jnp.transpose(w[:, :, 0, 0])


def _ws2(w):
    o, i = w.shape[0], w.shape[1]
    wt = jnp.transpose(w, (2, 3, 1, 0))
    we = jnp.zeros((2, 2, 2, 2, i, o), _F32)
    for dy in range(3):
        for dx in range(3):
            we = we.at[dy // 2, dx // 2, dy % 2, dx % 2].set(wt[dy, dx])
    return we.reshape(16 * i, o)


def _fc_fold(fc_w):
    iu = np.triu_indices(64)
    wf = jnp.zeros((64, 64, 128), _F32)
    wf = wf.at[iu[0], iu[1], :].set(jnp.transpose(fc_w))
    return jnp.transpose(wf, (1, 0, 2)).reshape(64, 64 * 128)


def _s2d(y):
    b, h, w, c = y.shape
    return (y.reshape(b, h // 2, 2, w // 2, 2, c)
             .transpose(0, 1, 3, 2, 4, 5)
             .reshape(b, h // 2, w // 2, 4 * c))


def _relu(x):
    return jnp.maximum(x, 0.0)


def _rbf(x):
    return x.astype(jnp.bfloat16).astype(_F32)


def _mmc(a, b):
    return jnp.dot(_rbf(a), _rbf(b), preferred_element_type=_F32)


def _mmh(a, b):
    return jnp.dot(a, b, preferred_element_type=_F32)


def _conv3(x, wref):
    h, w, c = x.shape
    xp = jnp.pad(x, ((1, 1), (1, 1), (0, 0)))
    cols = jnp.concatenate(
        [xp[dy:dy + h, dx:dx + w, :] for dy in range(3) for dx in range(3)],
        axis=2)
    return _mmc(cols.reshape(h * w, 9 * c), wref[:])


def _conv_s2(x, wref):
    h, w, c = x.shape
    xp = jnp.pad(x, ((0, 1), (0, 1), (0, 0)))
    cols = jnp.concatenate(
        [xp[t:t + h, s:s + w, :] for t in range(2) for s in range(2)],
        axis=2)
    return _mmc(cols.reshape(h * w, 4 * c), wref[:])


def _sqrtm_ns(a, n):
    i0 = lax.broadcasted_iota(jnp.int32, (n, n), 0)
    i1 = lax.broadcasted_iota(jnp.int32, (n, n), 1)
    eye = jnp.where(i0 == i1, 1.0, 0.0).astype(_F32)
    i3 = 3.0 * eye
    tr = jnp.sum(a * eye, axis=(0, 1), keepdims=True)
    an = a / tr
    zy = 0.5 * (i3 - an)
    y = _mmh(an, zy)
    z = zy
    for _ in range(3):
        t = 0.5 * (i3 - _mmh(z, y))
        y, z = _mmh(y, t), _mmh(t, z)
    yzy = 0.5 * _mmh(y, i3 - _mmh(z, y))
    return yzy * jnp.sqrt(tr)


def _store_s2d(out_ref, h, s, c):
    h6 = h.reshape(s // 2, 2, s // 2, 2, c)
    for hp in range(2):
        for wp in range(2):
            out_ref[0, :, :, (hp * 2 + wp) * c:(hp * 2 + wp + 1) * c] = (
                h6[:, hp, :, wp, :])


def _cbr(x, wref, ab, relu=True):
    y = _conv3(x, wref) * ab[0][:] + ab[1][:]
    return _relu(y) if relu else y


def _stage1_body(x_ref, wc1, a0, b0, w11, a1, b1, w12, a2, b2,
                 w21, a3, b3, w22, a4, b4, out_ref):
    x = x_ref[0]
    h = _cbr(x, wc1, (a0, b0))
    t = _cbr(h.reshape(64, 64, 16), w11, (a1, b1))
    t = _cbr(t.reshape(64, 64, 16), w12, (a2, b2), relu=False)
    h = _relu(t + h)
    t = _cbr(h.reshape(64, 64, 16), w21, (a3, b3))
    t = _cbr(t.reshape(64, 64, 16), w22, (a4, b4), relu=False)
    h = _relu(t + h)
    _store_s2d(out_ref, h, 64, 16)


def _mid_body(x_ref, ws2, a0, b0, wd, ad, bd, w12, a2, b2,
              w21, a3, b3, w22, a4, b4, out_ref, *, s, c):
    x = x_ref[0]
    m = s * s
    cin4 = x.shape[2]
    c1 = _relu(_conv_s2(x, ws2) * a0[:] + b0[:])
    res = _mmc(x.reshape(m, cin4)[:, :cin4 // 4], wd[:]) * ad[:] + bd[:]
    t = _cbr(c1.reshape(s, s, c), w12, (a2, b2), relu=False)
    h = _relu(t + res)
    t = _cbr(h.reshape(s, s, c), w21, (a3, b3))
    t = _cbr(t.reshape(s, s, c), w22, (a4, b4), relu=False)
    h = _relu(t + h)
    _store_s2d(out_ref, h, s, c)


def _stage4_body(x_ref, ws2, a0, b0, wd, ad, bd, w12, a2, b2,
                 w21, a3, b3, w22, a4, b4,
                 wred, ared, bred, watt, aatt, batt, wfc, bfc, out_ref):
    x = x_ref[0]
    c1 = _relu(_conv_s2(x, ws2) * a0[:] + b0[:])
    res = _mmc(x.reshape(64, 256)[:, :64], wd[:]) * ad[:] + bd[:]
    t = _cbr(c1.reshape(8, 8, 128), w12, (a2, b2), relu=False)
    h = _relu(t + res)
    t = _cbr(h.reshape(8, 8, 128), w21, (a3, b3))
    t = _cbr(t.reshape(8, 8, 128), w22, (a4, b4), relu=False)
    h = _relu(t + h)
    z = _relu(_mmc(h, wred[:]) * ared[:] + bred[:])
    a = _relu(_mmc(z, watt[:]) * aatt[:] + batt[:])
    ac = a - jnp.mean(a, axis=0, keepdims=True)
    cov = lax.dot_general(ac, ac, (((0,), (0,)), ((), ())),
                          preferred_element_type=_F32) * (1.0 / 64.0)
    ya = _sqrtm_ns(cov, 64)
    u = _mmh(ya, wfc[:])
    parts = [u[i:i + 1, 128 * i:128 * (i + 1)] for i in range(64)]
    while len(parts) > 1:
        parts = [parts[i] + parts[i + 1] for i in range(0, len(parts), 2)]
    gate = jax.nn.sigmoid(parts[0] + bfc[:])
    zg = z * gate
    zc = zg - jnp.mean(zg, axis=0, keepdims=True)
    cov2 = lax.dot_general(zc, zc, (((0,), (0,)), ((), ())),
                           preferred_element_type=_F32) * (1.0 / 64.0)
    out_ref[0] = _sqrtm_ns(cov2, 128)


def _full_spec(shape):
    nd = len(shape)
    return pl.BlockSpec(shape, lambda b, _nd=nd: (0,) * _nd)


def _item_spec(shape):
    nd = len(shape)
    return pl.BlockSpec((1,) + shape,
                        lambda b, _nd=nd: (b,) + (0,) * _nd)


def _run_stage(body, x, ws, out_elem_shape):
    b = x.shape[0]
    in_specs = [_item_spec(x.shape[1:])] + [_full_spec(w.shape) for w in ws]
    return pl.pallas_call(
        body,
        grid=(b,),
        in_specs=in_specs,
        out_specs=_item_spec(out_elem_shape),
        out_shape=jax.ShapeDtypeStruct((b,) + out_elem_shape, _F32),
        compiler_params=pltpu.CompilerParams(
            dimension_semantics=("parallel",),
            vmem_limit_bytes=56 * 1024 * 1024,
        ),
    )(x, *ws)


def kernel(x, params):
    p = params
    b = x.shape[0]
    l1, l2, l3, l4 = p['layers']
    xh = jnp.transpose(x, (0, 2, 3, 1))

    ws1 = [_w3(p['conv1']), *_bn_ab(p['bn1'])]
    for blk in l1:
        for wk, bk in (('w1', 'bn1'), ('w2', 'bn2')):
            ws1 += [_w3(blk[wk]), *_bn_ab(blk[bk])]
    a1 = _run_stage(_stage1_body, xh, ws1, (32, 32, 64))

    def mid_ws(blk0, blk1):
        ws = [_ws2(blk0['w1']), *_bn_ab(blk0['bn1'])]
        ws += [_w1(blk0['dw']), *_bn_ab(blk0['dbn'])]
        ws += [_w3(blk0['w2']), *_bn_ab(blk0['bn2'])]
        for wk, bk in (('w1', 'bn1'), ('w2', 'bn2')):
            ws += [_w3(blk1[wk]), *_bn_ab(blk1[bk])]
        return ws

    a2 = _run_stage(functools.partial(_mid_body, s=32, c=32),
                    a1, mid_ws(l2[0], l2[1]), (16, 16, 128))
    a3 = _run_stage(functools.partial(_mid_body, s=16, c=64),
                    a2, mid_ws(l3[0], l3[1]), (8, 8, 256))

    ws4 = mid_ws(l4[0], l4[1])
    ws4 += [_w1(p['reduce_w']), *_bn_ab(p['reduce_bn'])]
    ws4 += [_w1(p['att']['w']), *_bn_ab(p['att']['bn'])]
    ws4 += [_fc_fold(p['att']['fc_w']), p['att']['fc_b'][None, :]]
    y = _run_stage(_stage4_body, a3, ws4, (128, 128))

    iu = np.triu_indices(128)
    flat = jnp.asarray(iu[0] * 128 + iu[1], jnp.int32)
    return y.reshape(b, 128 * 128)[:, flat]


# speedup vs baseline: 1.2954x; 1.2440x over previous
import functools

import numpy as np
import jax
import jax.numpy as jnp
from jax import lax
from jax.experimental import pallas as pl
from jax.experimental.pallas import tpu as pltpu

_EPS = 1e-5
_F32 = jnp.float32


def _bn_ab(bn):
    g, b, m, v = bn
    inv = g / jnp.sqrt(v + _EPS)
    return inv[None, :], (b - m * inv)[None, :]


def _w3(w):
    o, i = w.shape[0], w.shape[1]
    return jnp.transpose(w, (2, 3, 1, 0)).reshape(9 * i, o)


def _w1(w):
    return jnp.transpose(w[:, :, 0, 0])


def _ws2(w):
    o, i = w.shape[0], w.shape[1]
    wt = jnp.transpose(w, (2, 3, 1, 0))
    we = jnp.zeros((2, 2, 2, 2, i, o), _F32)
    for dy in range(3):
        for dx in range(3):
            we = we.at[dy // 2, dx // 2, dy % 2, dx % 2].set(wt[dy, dx])
    return we.reshape(16 * i, o)


def _fc_fold(fc_w):
    iu = np.triu_indices(64)
    wf = jnp.zeros((64, 64, 128), _F32)
    wf = wf.at[iu[0], iu[1], :].set(jnp.transpose(fc_w))
    return jnp.transpose(wf, (1, 0, 2)).reshape(64, 64 * 128)


def _s2d(y):
    b, h, w, c = y.shape
    return (y.reshape(b, h // 2, 2, w // 2, 2, c)
             .transpose(0, 1, 3, 2, 4, 5)
             .reshape(b, h // 2, w // 2, 4 * c))


def _relu(x):
    return jnp.maximum(x, 0.0)


def _rbf(x):
    return x.astype(jnp.bfloat16).astype(_F32)


def _mmc(a, b):
    return jnp.dot(_rbf(a), _rbf(b), preferred_element_type=_F32)


def _mmh(a, b):
    return jnp.dot(a, b, preferred_element_type=_F32)


def _conv3(x, wref):
    h, w, c = x.shape
    xp = jnp.pad(x, ((1, 1), (1, 1), (0, 0)))
    cols = jnp.concatenate(
        [xp[dy:dy + h, dx:dx + w, :] for dy in range(3) for dx in range(3)],
        axis=2)
    return _mmc(cols.reshape(h * w, 9 * c), wref[:])


def _conv_s2(x, wref):
    h, w, c = x.shape
    xp = jnp.pad(x, ((0, 1), (0, 1), (0, 0)))
    cols = jnp.concatenate(
        [xp[t:t + h, s:s + w, :] for t in range(2) for s in range(2)],
        axis=2)
    return _mmc(cols.reshape(h * w, 4 * c), wref[:])


def _sqrtm_ns(a, n):
    i0 = lax.broadcasted_iota(jnp.int32, (n, n), 0)
    i1 = lax.broadcasted_iota(jnp.int32, (n, n), 1)
    eye = jnp.where(i0 == i1, 1.0, 0.0).astype(_F32)
    i3 = 3.0 * eye
    tr = jnp.sum(a * eye, axis=(0, 1), keepdims=True)
    an = a / tr
    zy = 0.5 * (i3 - an)
    y = _mmh(an, zy)
    z = zy
    for _ in range(3):
        t = 0.5 * (i3 - _mmh(z, y))
        y, z = _mmh(y, t), _mmh(t, z)
    yzy = 0.5 * _mmh(y, i3 - _mmh(z, y))
    return yzy * jnp.sqrt(tr)


def _store_s2d(out_ref, h, s, c):
    h6 = h.reshape(s // 2, 2, s // 2, 2, c)
    for hp in range(2):
        for wp in range(2):
            out_ref[0, :, :, (hp * 2 + wp) * c:(hp * 2 + wp + 1) * c] = (
                h6[:, hp, :, wp, :])


def _cbr(x, wref, ab, relu=True):
    y = _conv3(x, wref) * ab[0][:] + ab[1][:]
    return _relu(y) if relu else y


def _stage1_body(x_ref, wc1, a0, b0, w11, a1, b1, w12, a2, b2,
                 w21, a3, b3, w22, a4, b4, out_ref):
    x = jnp.transpose(x_ref[0], (1, 2, 0))
    h = _cbr(x, wc1, (a0, b0))
    t = _cbr(h.reshape(64, 64, 16), w11, (a1, b1))
    t = _cbr(t.reshape(64, 64, 16), w12, (a2, b2), relu=False)
    h = _relu(t + h)
    t = _cbr(h.reshape(64, 64, 16), w21, (a3, b3))
    t = _cbr(t.reshape(64, 64, 16), w22, (a4, b4), relu=False)
    h = _relu(t + h)
    _store_s2d(out_ref, h, 64, 16)


def _mid_body(x_ref, ws2, a0, b0, wd, ad, bd, w12, a2, b2,
              w21, a3, b3, w22, a4, b4, out_ref, *, s, c):
    x = x_ref[0]
    m = s * s
    cin4 = x.shape[2]
    c1 = _relu(_conv_s2(x, ws2) * a0[:] + b0[:])
    res = _mmc(x.reshape(m, cin4)[:, :cin4 // 4], wd[:]) * ad[:] + bd[:]
    t = _cbr(c1.reshape(s, s, c), w12, (a2, b2), relu=False)
    h = _relu(t + res)
    t = _cbr(h.reshape(s, s, c), w21, (a3, b3))
    t = _cbr(t.reshape(s, s, c), w22, (a4, b4), relu=False)
    h = _relu(t + h)
    _store_s2d(out_ref, h, s, c)


def _stage4_body(x_ref, ws2, a0, b0, wd, ad, bd, w12, a2, b2,
                 w21, a3, b3, w22, a4, b4,
                 wred, ared, bred, watt, aatt, batt, wfc, bfc, out_ref):
    x = x_ref[0]
    c1 = _relu(_conv_s2(x, ws2) * a0[:] + b0[:])
    res = _mmc(x.reshape(64, 256)[:, :64], wd[:]) * ad[:] + bd[:]
    t = _cbr(c1.reshape(8, 8, 128), w12, (a2, b2), relu=False)
    h = _relu(t + res)
    t = _cbr(h.reshape(8, 8, 128), w21, (a3, b3))
    t = _cbr(t.reshape(8, 8, 128), w22, (a4, b4), relu=False)
    h = _relu(t + h)
    z = _relu(_mmc(h, wred[:]) * ared[:] + bred[:])
    a = _relu(_mmc(z, watt[:]) * aatt[:] + batt[:])
    ac = a - jnp.mean(a, axis=0, keepdims=True)
    cov = lax.dot_general(ac, ac, (((0,), (0,)), ((), ())),
                          preferred_element_type=_F32) * (1.0 / 64.0)
    ya = _sqrtm_ns(cov, 64)
    u = _mmh(ya, wfc[:])
    parts = [u[i:i + 1, 128 * i:128 * (i + 1)] for i in range(64)]
    while len(parts) > 1:
        parts = [parts[i] + parts[i + 1] for i in range(0, len(parts), 2)]
    gate = jax.nn.sigmoid(parts[0] + bfc[:])
    zg = z * gate
    zc = zg - jnp.mean(zg, axis=0, keepdims=True)
    cov2 = lax.dot_general(zc, zc, (((0,), (0,)), ((), ())),
                           preferred_element_type=_F32) * (1.0 / 64.0)
    out_ref[0] = _sqrtm_ns(cov2, 128)


def _full_spec(shape):
    nd = len(shape)
    return pl.BlockSpec(shape, lambda b, _nd=nd: (0,) * _nd)


def _item_spec(shape):
    nd = len(shape)
    return pl.BlockSpec((1,) + shape,
                        lambda b, _nd=nd: (b,) + (0,) * _nd)


def _run_stage(body, x, ws, out_elem_shape):
    b = x.shape[0]
    in_specs = [_item_spec(x.shape[1:])] + [_full_spec(w.shape) for w in ws]
    return pl.pallas_call(
        body,
        grid=(b,),
        in_specs=in_specs,
        out_specs=_item_spec(out_elem_shape),
        out_shape=jax.ShapeDtypeStruct((b,) + out_elem_shape, _F32),
        compiler_params=pltpu.CompilerParams(
            dimension_semantics=("parallel",),
            vmem_limit_bytes=56 * 1024 * 1024,
        ),
    )(x, *ws)


def kernel(x, params):
    p = params
    b = x.shape[0]
    l1, l2, l3, l4 = p['layers']
    xh = x
    ws1 = [_w3(p['conv1']), *_bn_ab(p['bn1'])]
    for blk in l1:
        for wk, bk in (('w1', 'bn1'), ('w2', 'bn2')):
            ws1 += [_w3(blk[wk]), *_bn_ab(blk[bk])]
    a1 = _run_stage(_stage1_body, xh, ws1, (32, 32, 64))

    def mid_ws(blk0, blk1):
        ws = [_ws2(blk0['w1']), *_bn_ab(blk0['bn1'])]
        ws += [_w1(blk0['dw']), *_bn_ab(blk0['dbn'])]
        ws += [_w3(blk0['w2']), *_bn_ab(blk0['bn2'])]
        for wk, bk in (('w1', 'bn1'), ('w2', 'bn2')):
            ws += [_w3(blk1[wk]), *_bn_ab(blk1[bk])]
        return ws

    a2 = _run_stage(functools.partial(_mid_body, s=32, c=32),
                    a1, mid_ws(l2[0], l2[1]), (16, 16, 128))
    a3 = _run_stage(functools.partial(_mid_body, s=16, c=64),
                    a2, mid_ws(l3[0], l3[1]), (8, 8, 256))

    ws4 = mid_ws(l4[0], l4[1])
    ws4 += [_w1(p['reduce_w']), *_bn_ab(p['reduce_bn'])]
    ws4 += [_w1(p['att']['w']), *_bn_ab(p['att']['bn'])]
    ws4 += [_fc_fold(p['att']['fc_w']), p['att']['fc_b'][None, :]]
    y = _run_stage(_stage4_body, a3, ws4, (128, 128))

    iu = np.triu_indices(128)
    flat = jnp.asarray(iu[0] * 128 + iu[1], jnp.int32)
    return y.reshape(b, 128 * 128)[:, flat]
